# baseline (speedup 1.0000x reference)
import functools

import jax
import jax.numpy as jnp
from jax import lax
from jax.experimental import pallas as pl
from jax.experimental.pallas import tpu as pltpu
from jax.experimental.pallas import tpu_sc as plsc

N, D, H, E, K = 2048, 1024, 2048, 8, 2
A = N * K
BLK = 512
TMAX = 15
P = TMAX * BLK
GR, AW = A // 128, 128
NC, NS = 2, 16
NW = NC * NS
TE_PAD = 64
BN = 512


def _router_body(x_ref, g_ref, b_ref, wr_ref, ln2_ref, idx_ref, gate_ref):
    x = x_ref[...]
    m = jnp.mean(x, axis=-1, keepdims=True)
    v = jnp.mean((x - m) ** 2, axis=-1, keepdims=True)
    xf = (x - m) * lax.rsqrt(v + 1e-5) * g_ref[...] + b_ref[...]
    logits = lax.dot_general(xf, wr_ref[...], (((1,), (1,)), ((), ())),
                             preferred_element_type=jnp.float32)
    m2 = jnp.mean(xf, axis=-1, keepdims=True)
    v2 = jnp.mean((xf - m2) ** 2, axis=-1, keepdims=True)
    ln2_ref[...] = (xf - m2) * lax.rsqrt(v2 + 1e-5)
    ie = lax.broadcasted_iota(jnp.int32, logits.shape, 1)
    t1 = jnp.max(logits, axis=-1, keepdims=True)
    i1 = jnp.min(jnp.where(logits == t1, ie, E), axis=-1, keepdims=True)
    masked = jnp.where(ie == i1, -jnp.inf, logits)
    t2 = jnp.max(masked, axis=-1, keepdims=True)
    i2 = jnp.min(jnp.where(masked == t2, ie, E), axis=-1, keepdims=True)
    g1 = 1.0 / (1.0 + jnp.exp(t2 - t1))
    idx_ref[...] = jnp.concatenate([i1, i2], axis=1)
    gate_ref[...] = jnp.concatenate([g1, 1.0 - g1], axis=1)


def _router(xf, ln_g, ln_b, Wr):
    return pl.pallas_call(
        _router_body,
        grid=(N // BN,),
        in_specs=[
            pl.BlockSpec((BN, D), lambda i: (i, 0)),
            pl.BlockSpec((1, D), lambda i: (0, 0)),
            pl.BlockSpec((1, D), lambda i: (0, 0)),
            pl.BlockSpec((E, D), lambda i: (0, 0)),
        ],
        out_specs=[
            pl.BlockSpec((BN, D), lambda i: (i, 0)),
            pl.BlockSpec((BN, K), lambda i: (i, 0)),
            pl.BlockSpec((BN, K), lambda i: (i, 0)),
        ],
        out_shape=[
            jax.ShapeDtypeStruct((N, D), jnp.float32),
            jax.ShapeDtypeStruct((N, K), jnp.int32),
            jax.ShapeDtypeStruct((N, K), jnp.float32),
        ],
    )(xf, ln_g, ln_b, Wr)


def _plan_body(idx_ref, dest_ref, te_ref, nt_ref):
    idx2 = idx_ref[...]
    fi = lax.broadcasted_iota(jnp.int32, (AW, AW), 0)
    fj = lax.broadcasted_iota(jnp.int32, (AW, AW), 1)
    U = (fi < fj).astype(jnp.float32)
    gi = lax.broadcasted_iota(jnp.int32, (GR, GR), 0)
    gj = lax.broadcasted_iota(jnp.int32, (GR, GR), 1)
    Ls = (gj < gi).astype(jnp.float32)
    ohs = [(idx2 == e).astype(jnp.float32) for e in range(E)]
    S = jnp.concatenate(
        [jnp.sum(oh, axis=1, keepdims=True) for oh in ohs], axis=1)
    Pm = lax.dot_general(Ls, S, (((1,), (0,)), ((), ())),
                         preferred_element_type=jnp.float32)
    c = jnp.sum(S, axis=0, keepdims=True)
    nt_e = jnp.ceil(c * (1.0 / BLK))
    ei = lax.broadcasted_iota(jnp.int32, (E, E), 0)
    ej = lax.broadcasted_iota(jnp.int32, (E, E), 1)
    UE = (ei < ej).astype(jnp.float32)
    off_t = lax.dot_general(nt_e, UE, (((1,), (0,)), ((), ())),
                            preferred_element_type=jnp.float32)
    total = jnp.sum(nt_e, axis=1, keepdims=True)
    rank = jnp.zeros((GR, AW), jnp.float32)
    for e in range(E):
        cum = lax.dot_general(ohs[e], U, (((1,), (0,)), ((), ())),
                              preferred_element_type=jnp.float32)
        base = off_t[0:1, e:e + 1] * float(BLK)
        rank = rank + ohs[e] * (cum + Pm[:, e:e + 1] + base)
    dest_ref[...] = rank.astype(jnp.int32)
    tio = lax.broadcasted_iota(jnp.int32, (1, TE_PAD), 1).astype(jnp.float32)
    tcl = jnp.minimum(tio, total - 1.0)
    endk = off_t + nt_e
    te = jnp.zeros((1, TE_PAD), jnp.float32)
    for e in range(E):
        te = te + (tcl >= endk[0:1, e:e + 1]).astype(jnp.float32)
    te_ref[...] = te.astype(jnp.int32)
    nt_ref[...] = total.astype(jnp.int32)


def _plan(idxT):
    return pl.pallas_call(
        _plan_body,
        out_shape=[
            jax.ShapeDtypeStruct((GR, AW), jnp.int32),
            jax.ShapeDtypeStruct((1, TE_PAD), jnp.int32),
            jax.ShapeDtypeStruct((1, 1), jnp.int32),
        ],
    )(idxT)


ASG_W = A // NW
DCH = 32

@functools.cache
def _sc_mesh():
    return plsc.VectorSubcoreMesh(
        core_axis_name="c", subcore_axis_name="s",
        num_cores=NC, num_subcores=NS)


def _dispatch_body(ln2_hbm, dest2_hbm, xs_hbm, idx_m, rb0, rb1, semL0, semL1,
                   semS0, semS1):
    wid = lax.axis_index("s") * NC + lax.axis_index("c")
    nch = ASG_W // DCH
    rowbase = wid * nch
    tbase = (wid * ASG_W) % N
    pltpu.sync_copy(dest2_hbm.at[pl.ds(rowbase, nch)], idx_m)
    rb = (rb0, rb1)
    semL = (semL0, semL1)
    semS = (semS0, semS1)
    ld = [None] * nch
    sc = [None] * nch
    ld[0] = pltpu.async_copy(ln2_hbm.at[pl.ds(tbase, DCH)], rb[0], semL[0])
    for c in range(nch):
        ld[c].wait()
        sc[c] = pltpu.async_copy(rb[c % 2], xs_hbm.at[idx_m.at[c]], semS[c % 2])
        if c + 1 < nch:
            if c >= 1:
                sc[c - 1].wait()
            ld[c + 1] = pltpu.async_copy(
                ln2_hbm.at[pl.ds(tbase + (c + 1) * DCH, DCH)],
                rb[(c + 1) % 2], semL[(c + 1) % 2])
    sc[nch - 2].wait()
    sc[nch - 1].wait()


def _dispatch(ln2, destf):
    f = pl.kernel(
        _dispatch_body,
        out_type=jax.ShapeDtypeStruct((P, D), jnp.float32),
        mesh=_sc_mesh(),
        scratch_types=[
            pltpu.VMEM((ASG_W // DCH, DCH), jnp.int32),
            pltpu.VMEM((DCH, D), jnp.float32),
            pltpu.VMEM((DCH, D), jnp.float32),
            pltpu.SemaphoreType.DMA,
            pltpu.SemaphoreType.DMA,
            pltpu.SemaphoreType.DMA,
            pltpu.SemaphoreType.DMA,
        ],
    )
    return f(ln2, destf.reshape(A // DCH, DCH))


def _gmm_body(nt_s, te_s, xs_ref, w1_ref, w2_ref, wo_ref, eng_ref, enb_ref,
              b1_ref, b2_ref, bo_ref, y_ref):
    t = pl.program_id(0)

    @pl.when(t < nt_s[0])
    def _():
        h = (xs_ref[...] * eng_ref[0] + enb_ref[0]).astype(jnp.bfloat16)
        x1 = lax.dot_general(h, w1_ref[0], (((1,), (1,)), ((), ())),
                             preferred_element_type=jnp.float32) + b1_ref[0]
        x2 = lax.dot_general(h, w2_ref[0], (((1,), (1,)), ((), ())),
                             preferred_element_type=jnp.float32) + b2_ref[0]
        g = jnp.clip(x2, -20.0, 20.0)
        g = g / (1.0 + jnp.exp(-g))
        hid = jnp.clip(x1 * g, -10000.0, 10000.0).astype(jnp.bfloat16)
        y = lax.dot_general(hid, wo_ref[0], (((1,), (1,)), ((), ())),
                            preferred_element_type=jnp.float32) + bo_ref[0]
        y_ref[...] = jnp.clip(y, -10000.0, 10000.0)


def _gmm(ntiles, te, xs, W1, W2, Wo, eng, enb, b1, b2, bo):
    def row_idx(t, nt, te):
        return (jnp.minimum(t, nt[0] - 1), 0)

    def exp_idx3(t, nt, te):
        return (te[jnp.minimum(t, nt[0] - 1)], 0, 0)

    def exp_idx2(t, nt, te):
        return (te[jnp.minimum(t, nt[0] - 1)], 0)

    grid_spec = pltpu.PrefetchScalarGridSpec(
        num_scalar_prefetch=2,
        grid=(TMAX,),
        in_specs=[
            pl.BlockSpec((BLK, D), row_idx),
            pl.BlockSpec((1, H, D), exp_idx3),
            pl.BlockSpec((1, H, D), exp_idx3),
            pl.BlockSpec((1, D, H), exp_idx3),
            pl.BlockSpec((1, 1, D), exp_idx3),
            pl.BlockSpec((1, 1, D), exp_idx3),
            pl.BlockSpec((1, 1, H), exp_idx3),
            pl.BlockSpec((1, 1, H), exp_idx3),
            pl.BlockSpec((1, 1, D), exp_idx3),
        ],
        out_specs=pl.BlockSpec((BLK, D), row_idx),
    )
    return pl.pallas_call(
        _gmm_body,
        grid_spec=grid_spec,
        out_shape=jax.ShapeDtypeStruct((P, D), jnp.float32),
        compiler_params=pltpu.CompilerParams(vmem_limit_bytes=120 * 2**20),
    )(ntiles, te, xs, W1, W2, Wo,
      eng.reshape(E, 1, D), enb.reshape(E, 1, D),
      b1.reshape(E, 1, H), b2.reshape(E, 1, H), bo.reshape(E, 1, D))


TOK_W = N // NW
CCH = 16


def _combine_body(y_hbm, dest2_hbm, z0_hbm, z1_hbm, d_m,
                  r0a, r0b, r1a, r1b, semG0, semG1, semW0, semW1):
    wid = lax.axis_index("s") * NC + lax.axis_index("c")
    nch = TOK_W // CCH
    rbase = wid * nch
    pltpu.sync_copy(dest2_hbm.at[pl.ds(rbase, nch)], d_m.at[pl.ds(0, nch)])
    pltpu.sync_copy(dest2_hbm.at[pl.ds(N // CCH + rbase, nch)],
                    d_m.at[pl.ds(nch, nch)])
    r0 = (r0a, r0b)
    r1 = (r1a, r1b)
    semG = (semG0, semG1)
    semW = (semW0, semW1)
    gt = [None] * nch
    wr = [None] * nch
    gt[0] = (pltpu.async_copy(y_hbm.at[d_m.at[0]], r0[0], semG[0]),
             pltpu.async_copy(y_hbm.at[d_m.at[nch]], r1[0], semG[0]))
    for c in range(nch):
        nb = wid * TOK_W + c * CCH
        gt[c][0].wait()
        gt[c][1].wait()
        wr[c] = (pltpu.async_copy(r0[c % 2], z0_hbm.at[pl.ds(nb, CCH)],
                                  semW[c % 2]),
                 pltpu.async_copy(r1[c % 2], z1_hbm.at[pl.ds(nb, CCH)],
                                  semW[c % 2]))
        if c + 1 < nch:
            if c >= 1:
                wr[c - 1][0].wait()
                wr[c - 1][1].wait()
            gt[c + 1] = (pltpu.async_copy(y_hbm.at[d_m.at[c + 1]],
                                          r0[(c + 1) % 2], semG[(c + 1) % 2]),
                         pltpu.async_copy(y_hbm.at[d_m.at[nch + c + 1]],
                                          r1[(c + 1) % 2], semG[(c + 1) % 2]))
    wr[nch - 2][0].wait()
    wr[nch - 2][1].wait()
    wr[nch - 1][0].wait()
    wr[nch - 1][1].wait()


def _combine_gather(y, destf):
    f = pl.kernel(
        _combine_body,
        out_type=[
            jax.ShapeDtypeStruct((N, D), jnp.float32),
            jax.ShapeDtypeStruct((N, D), jnp.float32),
        ],
        mesh=_sc_mesh(),
        scratch_types=[
            pltpu.VMEM((2 * (TOK_W // CCH), CCH), jnp.int32),
            pltpu.VMEM((CCH, D), jnp.float32),
            pltpu.VMEM((CCH, D), jnp.float32),
            pltpu.VMEM((CCH, D), jnp.float32),
            pltpu.VMEM((CCH, D), jnp.float32),
            pltpu.SemaphoreType.DMA,
            pltpu.SemaphoreType.DMA,
            pltpu.SemaphoreType.DMA,
            pltpu.SemaphoreType.DMA,
        ],
    )
    return f(y, destf.reshape(A // CCH, CCH))


def _blend_body(z0_ref, z1_ref, g0_ref, g1_ref, out_ref):
    out_ref[...] = g0_ref[...] * z0_ref[...] + g1_ref[...] * z1_ref[...]


def _blend(z0, z1, g0, g1):
    return pl.pallas_call(
        _blend_body,
        grid=(N // BN,),
        in_specs=[
            pl.BlockSpec((BN, D), lambda i: (i, 0)),
            pl.BlockSpec((BN, D), lambda i: (i, 0)),
            pl.BlockSpec((BN, 1), lambda i: (i, 0)),
            pl.BlockSpec((BN, 1), lambda i: (i, 0)),
        ],
        out_specs=pl.BlockSpec((BN, D), lambda i: (i, 0)),
        out_shape=jax.ShapeDtypeStruct((N, D), jnp.float32),
    )(z0, z1, g0, g1)


def kernel(x, ln_g, ln_b, Wr, eng, enb, W1, b1, W2, b2, Wo, bo):
    assert x.shape == (1, N, D)
    xf = x.reshape(N, D)
    ln2, idx, gate = _router(xf, ln_g.reshape(1, D), ln_b.reshape(1, D), Wr)
    idxT = idx.T.reshape(GR, AW)
    dest2, te2, nt2 = _plan(idxT)
    destf = dest2.reshape(A)
    xs = _dispatch(ln2, destf)
    y = _gmm(nt2.reshape(1), te2.reshape(TE_PAD), xs,
             W1.astype(jnp.bfloat16), W2.astype(jnp.bfloat16),
             Wo.astype(jnp.bfloat16), eng, enb, b1, b2, bo)
    z0, z1 = _combine_gather(y, destf)
    out = _blend(z0, z1, gate[:, 0:1], gate[:, 1:2])
    return out.reshape(x.shape)


# speedup vs baseline: 1.4720x; 1.4720x over previous
import functools

import jax
import jax.numpy as jnp
from jax import lax
from jax.experimental import pallas as pl
from jax.experimental.pallas import tpu as pltpu
from jax.experimental.pallas import tpu_sc as plsc

N, D, H, E, K = 2048, 1024, 2048, 8, 2
A = N * K
BLK = 512
TMAX = 15
P = TMAX * BLK
GR, AW = A // 128, 128
NC, NS = 2, 16
NW = NC * NS
TE_PAD = 64
BN = 512


def _router_body(x_ref, g_ref, b_ref, wr_ref, ln2_ref, idx_ref, gate_ref):
    x = x_ref[...]
    m = jnp.mean(x, axis=-1, keepdims=True)
    v = jnp.mean((x - m) ** 2, axis=-1, keepdims=True)
    xf = (x - m) * lax.rsqrt(v + 1e-5) * g_ref[...] + b_ref[...]
    logits = lax.dot_general(xf, wr_ref[...], (((1,), (1,)), ((), ())),
                             preferred_element_type=jnp.float32)
    m2 = jnp.mean(xf, axis=-1, keepdims=True)
    v2 = jnp.mean((xf - m2) ** 2, axis=-1, keepdims=True)
    ln2_ref[...] = (xf - m2) * lax.rsqrt(v2 + 1e-5)
    ie = lax.broadcasted_iota(jnp.int32, logits.shape, 1)
    t1 = jnp.max(logits, axis=-1, keepdims=True)
    i1 = jnp.min(jnp.where(logits == t1, ie, E), axis=-1, keepdims=True)
    masked = jnp.where(ie == i1, -jnp.inf, logits)
    t2 = jnp.max(masked, axis=-1, keepdims=True)
    i2 = jnp.min(jnp.where(masked == t2, ie, E), axis=-1, keepdims=True)
    g1 = 1.0 / (1.0 + jnp.exp(t2 - t1))
    idx_ref[...] = jnp.concatenate([i1, i2], axis=1)
    gate_ref[...] = jnp.concatenate([g1, 1.0 - g1], axis=1)


def _router(xf, ln_g, ln_b, Wr):
    return pl.pallas_call(
        _router_body,
        grid=(N // BN,),
        in_specs=[
            pl.BlockSpec((BN, D), lambda i: (i, 0)),
            pl.BlockSpec((1, D), lambda i: (0, 0)),
            pl.BlockSpec((1, D), lambda i: (0, 0)),
            pl.BlockSpec((E, D), lambda i: (0, 0)),
        ],
        out_specs=[
            pl.BlockSpec((BN, D), lambda i: (i, 0)),
            pl.BlockSpec((BN, K), lambda i: (i, 0)),
            pl.BlockSpec((BN, K), lambda i: (i, 0)),
        ],
        out_shape=[
            jax.ShapeDtypeStruct((N, D), jnp.float32),
            jax.ShapeDtypeStruct((N, K), jnp.int32),
            jax.ShapeDtypeStruct((N, K), jnp.float32),
        ],
    )(xf, ln_g, ln_b, Wr)


def _plan_body(idx_ref, dest_ref, te_ref, nt_ref):
    idx2 = idx_ref[...]
    fi = lax.broadcasted_iota(jnp.int32, (AW, AW), 0)
    fj = lax.broadcasted_iota(jnp.int32, (AW, AW), 1)
    U = (fi < fj).astype(jnp.float32)
    gi = lax.broadcasted_iota(jnp.int32, (GR, GR), 0)
    gj = lax.broadcasted_iota(jnp.int32, (GR, GR), 1)
    Ls = (gj < gi).astype(jnp.float32)
    ohs = [(idx2 == e).astype(jnp.float32) for e in range(E)]
    S = jnp.concatenate(
        [jnp.sum(oh, axis=1, keepdims=True) for oh in ohs], axis=1)
    Pm = lax.dot_general(Ls, S, (((1,), (0,)), ((), ())),
                         preferred_element_type=jnp.float32)
    c = jnp.sum(S, axis=0, keepdims=True)
    nt_e = jnp.ceil(c * (1.0 / BLK))
    ei = lax.broadcasted_iota(jnp.int32, (E, E), 0)
    ej = lax.broadcasted_iota(jnp.int32, (E, E), 1)
    UE = (ei < ej).astype(jnp.float32)
    off_t = lax.dot_general(nt_e, UE, (((1,), (0,)), ((), ())),
                            preferred_element_type=jnp.float32)
    total = jnp.sum(nt_e, axis=1, keepdims=True)
    rank = jnp.zeros((GR, AW), jnp.float32)
    for e in range(E):
        cum = lax.dot_general(ohs[e], U, (((1,), (0,)), ((), ())),
                              preferred_element_type=jnp.float32)
        base = off_t[0:1, e:e + 1] * float(BLK)
        rank = rank + ohs[e] * (cum + Pm[:, e:e + 1] + base)
    dest_ref[...] = rank.astype(jnp.int32)
    tio = lax.broadcasted_iota(jnp.int32, (1, TE_PAD), 1).astype(jnp.float32)
    tcl = jnp.minimum(tio, total - 1.0)
    endk = off_t + nt_e
    te = jnp.zeros((1, TE_PAD), jnp.float32)
    for e in range(E):
        te = te + (tcl >= endk[0:1, e:e + 1]).astype(jnp.float32)
    te_ref[...] = te.astype(jnp.int32)
    nt_ref[...] = total.astype(jnp.int32)


def _plan(idxT):
    return pl.pallas_call(
        _plan_body,
        out_shape=[
            jax.ShapeDtypeStruct((GR, AW), jnp.int32),
            jax.ShapeDtypeStruct((1, TE_PAD), jnp.int32),
            jax.ShapeDtypeStruct((1, 1), jnp.int32),
        ],
    )(idxT)


ASG_W = A // NW
DCH = 32

@functools.cache
def _sc_mesh():
    return plsc.VectorSubcoreMesh(
        core_axis_name="c", subcore_axis_name="s",
        num_cores=NC, num_subcores=NS)


def _dispatch_body(ln2_hbm, dest2_hbm, xs_hbm, idx_m, rb0, rb1, semL0, semL1,
                   semS0, semS1):
    wid = lax.axis_index("s") * NC + lax.axis_index("c")
    nch = ASG_W // DCH
    rowbase = wid * nch
    tbase = (wid * ASG_W) % N
    pltpu.sync_copy(dest2_hbm.at[pl.ds(rowbase, nch)], idx_m)
    rb = (rb0, rb1)
    semL = (semL0, semL1)
    semS = (semS0, semS1)
    ld = [None] * nch
    sc = [None] * nch
    ld[0] = pltpu.async_copy(ln2_hbm.at[pl.ds(tbase, DCH)], rb[0], semL[0])
    for c in range(nch):
        ld[c].wait()
        sc[c] = pltpu.async_copy(rb[c % 2], xs_hbm.at[idx_m.at[c]], semS[c % 2])
        if c + 1 < nch:
            if c >= 1:
                sc[c - 1].wait()
            ld[c + 1] = pltpu.async_copy(
                ln2_hbm.at[pl.ds(tbase + (c + 1) * DCH, DCH)],
                rb[(c + 1) % 2], semL[(c + 1) % 2])
    sc[nch - 2].wait()
    sc[nch - 1].wait()


def _dispatch(ln2, destf):
    f = pl.kernel(
        _dispatch_body,
        out_type=jax.ShapeDtypeStruct((P, D), jnp.float32),
        mesh=_sc_mesh(),
        scratch_types=[
            pltpu.VMEM((ASG_W // DCH, DCH), jnp.int32),
            pltpu.VMEM((DCH, D), jnp.float32),
            pltpu.VMEM((DCH, D), jnp.float32),
            pltpu.SemaphoreType.DMA,
            pltpu.SemaphoreType.DMA,
            pltpu.SemaphoreType.DMA,
            pltpu.SemaphoreType.DMA,
        ],
    )
    return f(ln2, destf.reshape(A // DCH, DCH))


def _gmm_body(nt_s, te_s, xs_ref, w1_ref, w2_ref, wo_ref, eng_ref, enb_ref,
              b1_ref, b2_ref, bo_ref, y_ref):
    t = pl.program_id(0)

    @pl.when(t < nt_s[0])
    def _():
        h = xs_ref[...] * eng_ref[0] + enb_ref[0]
        x1 = lax.dot_general(h, w1_ref[0], (((1,), (1,)), ((), ())),
                             preferred_element_type=jnp.float32) + b1_ref[0]
        x2 = lax.dot_general(h, w2_ref[0], (((1,), (1,)), ((), ())),
                             preferred_element_type=jnp.float32) + b2_ref[0]
        g = jnp.clip(x2, -20.0, 20.0)
        g = g / (1.0 + jnp.exp(-g))
        hid = jnp.clip(x1 * g, -10000.0, 10000.0)
        y = lax.dot_general(hid, wo_ref[0], (((1,), (1,)), ((), ())),
                            preferred_element_type=jnp.float32) + bo_ref[0]
        y_ref[...] = jnp.clip(y, -10000.0, 10000.0)


def _gmm(ntiles, te, xs, W1, W2, Wo, eng, enb, b1, b2, bo):
    def row_idx(t, nt, te):
        return (jnp.minimum(t, nt[0] - 1), 0)

    def exp_idx3(t, nt, te):
        return (te[jnp.minimum(t, nt[0] - 1)], 0, 0)

    def exp_idx2(t, nt, te):
        return (te[jnp.minimum(t, nt[0] - 1)], 0)

    grid_spec = pltpu.PrefetchScalarGridSpec(
        num_scalar_prefetch=2,
        grid=(TMAX,),
        in_specs=[
            pl.BlockSpec((BLK, D), row_idx),
            pl.BlockSpec((1, H, D), exp_idx3),
            pl.BlockSpec((1, H, D), exp_idx3),
            pl.BlockSpec((1, D, H), exp_idx3),
            pl.BlockSpec((1, 1, D), exp_idx3),
            pl.BlockSpec((1, 1, D), exp_idx3),
            pl.BlockSpec((1, 1, H), exp_idx3),
            pl.BlockSpec((1, 1, H), exp_idx3),
            pl.BlockSpec((1, 1, D), exp_idx3),
        ],
        out_specs=pl.BlockSpec((BLK, D), row_idx),
    )
    return pl.pallas_call(
        _gmm_body,
        grid_spec=grid_spec,
        out_shape=jax.ShapeDtypeStruct((P, D), jnp.float32),
        compiler_params=pltpu.CompilerParams(vmem_limit_bytes=120 * 2**20),
    )(ntiles, te, xs, W1, W2, Wo,
      eng.reshape(E, 1, D), enb.reshape(E, 1, D),
      b1.reshape(E, 1, H), b2.reshape(E, 1, H), bo.reshape(E, 1, D))


TOK_W = N // NW
CCH = 16


def _combine_body(y_hbm, dest2_hbm, z0_hbm, z1_hbm, d_m,
                  r0a, r0b, r1a, r1b, semG0, semG1, semW0, semW1):
    wid = lax.axis_index("s") * NC + lax.axis_index("c")
    nch = TOK_W // CCH
    rbase = wid * nch
    pltpu.sync_copy(dest2_hbm.at[pl.ds(rbase, nch)], d_m.at[pl.ds(0, nch)])
    pltpu.sync_copy(dest2_hbm.at[pl.ds(N // CCH + rbase, nch)],
                    d_m.at[pl.ds(nch, nch)])
    r0 = (r0a, r0b)
    r1 = (r1a, r1b)
    semG = (semG0, semG1)
    semW = (semW0, semW1)
    gt = [None] * nch
    wr = [None] * nch
    gt[0] = (pltpu.async_copy(y_hbm.at[d_m.at[0]], r0[0], semG[0]),
             pltpu.async_copy(y_hbm.at[d_m.at[nch]], r1[0], semG[0]))
    for c in range(nch):
        nb = wid * TOK_W + c * CCH
        gt[c][0].wait()
        gt[c][1].wait()
        wr[c] = (pltpu.async_copy(r0[c % 2], z0_hbm.at[pl.ds(nb, CCH)],
                                  semW[c % 2]),
                 pltpu.async_copy(r1[c % 2], z1_hbm.at[pl.ds(nb, CCH)],
                                  semW[c % 2]))
        if c + 1 < nch:
            if c >= 1:
                wr[c - 1][0].wait()
                wr[c - 1][1].wait()
            gt[c + 1] = (pltpu.async_copy(y_hbm.at[d_m.at[c + 1]],
                                          r0[(c + 1) % 2], semG[(c + 1) % 2]),
                         pltpu.async_copy(y_hbm.at[d_m.at[nch + c + 1]],
                                          r1[(c + 1) % 2], semG[(c + 1) % 2]))
    wr[nch - 2][0].wait()
    wr[nch - 2][1].wait()
    wr[nch - 1][0].wait()
    wr[nch - 1][1].wait()


def _combine_gather(y, destf):
    f = pl.kernel(
        _combine_body,
        out_type=[
            jax.ShapeDtypeStruct((N, D), jnp.float32),
            jax.ShapeDtypeStruct((N, D), jnp.float32),
        ],
        mesh=_sc_mesh(),
        scratch_types=[
            pltpu.VMEM((2 * (TOK_W // CCH), CCH), jnp.int32),
            pltpu.VMEM((CCH, D), jnp.float32),
            pltpu.VMEM((CCH, D), jnp.float32),
            pltpu.VMEM((CCH, D), jnp.float32),
            pltpu.VMEM((CCH, D), jnp.float32),
            pltpu.SemaphoreType.DMA,
            pltpu.SemaphoreType.DMA,
            pltpu.SemaphoreType.DMA,
            pltpu.SemaphoreType.DMA,
        ],
    )
    return f(y, destf.reshape(A // CCH, CCH))


def _blend_body(z0_ref, z1_ref, g0_ref, g1_ref, out_ref):
    out_ref[...] = g0_ref[...] * z0_ref[...] + g1_ref[...] * z1_ref[...]


def _blend(z0, z1, g0, g1):
    return pl.pallas_call(
        _blend_body,
        grid=(N // BN,),
        in_specs=[
            pl.BlockSpec((BN, D), lambda i: (i, 0)),
            pl.BlockSpec((BN, D), lambda i: (i, 0)),
            pl.BlockSpec((BN, 1), lambda i: (i, 0)),
            pl.BlockSpec((BN, 1), lambda i: (i, 0)),
        ],
        out_specs=pl.BlockSpec((BN, D), lambda i: (i, 0)),
        out_shape=jax.ShapeDtypeStruct((N, D), jnp.float32),
    )(z0, z1, g0, g1)


def kernel(x, ln_g, ln_b, Wr, eng, enb, W1, b1, W2, b2, Wo, bo):
    assert x.shape == (1, N, D)
    xf = x.reshape(N, D)
    ln2, idx, gate = _router(xf, ln_g.reshape(1, D), ln_b.reshape(1, D), Wr)
    idxT = idx.T.reshape(GR, AW)
    dest2, te2, nt2 = _plan(idxT)
    destf = dest2.reshape(A)
    xs = _dispatch(ln2, destf)
    y = _gmm(nt2.reshape(1), te2.reshape(TE_PAD), xs,
             W1, W2, Wo, eng, enb, b1, b2, bo)
    z0, z1 = _combine_gather(y, destf)
    out = _blend(z0, z1, gate[:, 0:1], gate[:, 1:2])
    return out.reshape(x.shape)


# speedup vs baseline: 1.5843x; 1.0763x over previous
import functools

import jax
import jax.numpy as jnp
from jax import lax
from jax.experimental import pallas as pl
from jax.experimental.pallas import tpu as pltpu
from jax.experimental.pallas import tpu_sc as plsc

N, D, H, E, K = 2048, 1024, 2048, 8, 2
A = N * K
BLK = 536
TMAX = 15
P = TMAX * BLK
GR, AW = A // 128, 128
NC, NS = 2, 16
NW = NC * NS
TE_PAD = 64
BN = 512


def _router_body(x_ref, g_ref, b_ref, wr_ref, ln2_ref, idx_ref, gate_ref):
    x = x_ref[...]
    m = jnp.mean(x, axis=-1, keepdims=True)
    v = jnp.mean((x - m) ** 2, axis=-1, keepdims=True)
    xf = (x - m) * lax.rsqrt(v + 1e-5) * g_ref[...] + b_ref[...]
    logits = lax.dot_general(xf, wr_ref[...], (((1,), (1,)), ((), ())),
                             preferred_element_type=jnp.float32)
    m2 = jnp.mean(xf, axis=-1, keepdims=True)
    v2 = jnp.mean((xf - m2) ** 2, axis=-1, keepdims=True)
    ln2_ref[...] = (xf - m2) * lax.rsqrt(v2 + 1e-5)
    ie = lax.broadcasted_iota(jnp.int32, logits.shape, 1)
    t1 = jnp.max(logits, axis=-1, keepdims=True)
    i1 = jnp.min(jnp.where(logits == t1, ie, E), axis=-1, keepdims=True)
    masked = jnp.where(ie == i1, -jnp.inf, logits)
    t2 = jnp.max(masked, axis=-1, keepdims=True)
    i2 = jnp.min(jnp.where(masked == t2, ie, E), axis=-1, keepdims=True)
    g1 = 1.0 / (1.0 + jnp.exp(t2 - t1))
    idx_ref[...] = jnp.concatenate([i1, i2], axis=1)
    gate_ref[...] = jnp.concatenate([g1, 1.0 - g1], axis=1)


def _router(xf, ln_g, ln_b, Wr):
    return pl.pallas_call(
        _router_body,
        grid=(N // BN,),
        in_specs=[
            pl.BlockSpec((BN, D), lambda i: (i, 0)),
            pl.BlockSpec((1, D), lambda i: (0, 0)),
            pl.BlockSpec((1, D), lambda i: (0, 0)),
            pl.BlockSpec((E, D), lambda i: (0, 0)),
        ],
        out_specs=[
            pl.BlockSpec((BN, D), lambda i: (i, 0)),
            pl.BlockSpec((BN, K), lambda i: (i, 0)),
            pl.BlockSpec((BN, K), lambda i: (i, 0)),
        ],
        out_shape=[
            jax.ShapeDtypeStruct((N, D), jnp.float32),
            jax.ShapeDtypeStruct((N, K), jnp.int32),
            jax.ShapeDtypeStruct((N, K), jnp.float32),
        ],
    )(xf, ln_g, ln_b, Wr)


def _plan_body(idx_ref, dest_ref, te_ref, nt_ref):
    idx2 = idx_ref[...]
    fi = lax.broadcasted_iota(jnp.int32, (AW, AW), 0)
    fj = lax.broadcasted_iota(jnp.int32, (AW, AW), 1)
    U = (fi < fj).astype(jnp.float32)
    gi = lax.broadcasted_iota(jnp.int32, (GR, GR), 0)
    gj = lax.broadcasted_iota(jnp.int32, (GR, GR), 1)
    Ls = (gj < gi).astype(jnp.float32)
    ohs = [(idx2 == e).astype(jnp.float32) for e in range(E)]
    S = jnp.concatenate(
        [jnp.sum(oh, axis=1, keepdims=True) for oh in ohs], axis=1)
    Pm = lax.dot_general(Ls, S, (((1,), (0,)), ((), ())),
                         preferred_element_type=jnp.float32)
    c = jnp.sum(S, axis=0, keepdims=True)
    nt_e = jnp.ceil(c * (1.0 / BLK))
    ei = lax.broadcasted_iota(jnp.int32, (E, E), 0)
    ej = lax.broadcasted_iota(jnp.int32, (E, E), 1)
    UE = (ei < ej).astype(jnp.float32)
    off_t = lax.dot_general(nt_e, UE, (((1,), (0,)), ((), ())),
                            preferred_element_type=jnp.float32)
    total = jnp.sum(nt_e, axis=1, keepdims=True)
    rank = jnp.zeros((GR, AW), jnp.float32)
    for e in range(E):
        cum = lax.dot_general(ohs[e], U, (((1,), (0,)), ((), ())),
                              preferred_element_type=jnp.float32)
        base = off_t[0:1, e:e + 1] * float(BLK)
        rank = rank + ohs[e] * (cum + Pm[:, e:e + 1] + base)
    dest_ref[...] = rank.astype(jnp.int32)
    tio = lax.broadcasted_iota(jnp.int32, (1, TE_PAD), 1).astype(jnp.float32)
    tcl = jnp.minimum(tio, total - 1.0)
    endk = off_t + nt_e
    te = jnp.zeros((1, TE_PAD), jnp.float32)
    for e in range(E):
        te = te + (tcl >= endk[0:1, e:e + 1]).astype(jnp.float32)
    te_ref[...] = te.astype(jnp.int32)
    nt_ref[...] = total.astype(jnp.int32)


def _plan(idxT):
    return pl.pallas_call(
        _plan_body,
        out_shape=[
            jax.ShapeDtypeStruct((GR, AW), jnp.int32),
            jax.ShapeDtypeStruct((1, TE_PAD), jnp.int32),
            jax.ShapeDtypeStruct((1, 1), jnp.int32),
        ],
    )(idxT)


ASG_W = A // NW
DCH = 32

@functools.cache
def _sc_mesh():
    return plsc.VectorSubcoreMesh(
        core_axis_name="c", subcore_axis_name="s",
        num_cores=NC, num_subcores=NS)


def _dispatch_body(ln2_hbm, dest2_hbm, xs_hbm, idx_m, rb0, rb1, semL0, semL1,
                   semS0, semS1):
    wid = lax.axis_index("s") * NC + lax.axis_index("c")
    nch = ASG_W // DCH
    rowbase = wid * nch
    tbase = (wid * ASG_W) % N
    pltpu.sync_copy(dest2_hbm.at[pl.ds(rowbase, nch)], idx_m)
    rb = (rb0, rb1)
    semL = (semL0, semL1)
    semS = (semS0, semS1)
    ld = [None] * nch
    sc = [None] * nch
    ld[0] = pltpu.async_copy(ln2_hbm.at[pl.ds(tbase, DCH)], rb[0], semL[0])
    for c in range(nch):
        ld[c].wait()
        sc[c] = pltpu.async_copy(rb[c % 2], xs_hbm.at[idx_m.at[c]], semS[c % 2])
        if c + 1 < nch:
            if c >= 1:
                sc[c - 1].wait()
            ld[c + 1] = pltpu.async_copy(
                ln2_hbm.at[pl.ds(tbase + (c + 1) * DCH, DCH)],
                rb[(c + 1) % 2], semL[(c + 1) % 2])
    sc[nch - 2].wait()
    sc[nch - 1].wait()


def _dispatch(ln2, destf):
    f = pl.kernel(
        _dispatch_body,
        out_type=jax.ShapeDtypeStruct((P, D), jnp.float32),
        mesh=_sc_mesh(),
        scratch_types=[
            pltpu.VMEM((ASG_W // DCH, DCH), jnp.int32),
            pltpu.VMEM((DCH, D), jnp.float32),
            pltpu.VMEM((DCH, D), jnp.float32),
            pltpu.SemaphoreType.DMA,
            pltpu.SemaphoreType.DMA,
            pltpu.SemaphoreType.DMA,
            pltpu.SemaphoreType.DMA,
        ],
    )
    return f(ln2, destf.reshape(A // DCH, DCH))


def _gmm_body(nt_s, te_s, xs_ref, w1_ref, w2_ref, wo_ref, eng_ref, enb_ref,
              b1_ref, b2_ref, bo_ref, y_ref):
    t = pl.program_id(0)

    @pl.when(t < nt_s[0])
    def _():
        h = xs_ref[...] * eng_ref[0] + enb_ref[0]
        x1 = lax.dot_general(h, w1_ref[0], (((1,), (1,)), ((), ())),
                             preferred_element_type=jnp.float32) + b1_ref[0]
        x2 = lax.dot_general(h, w2_ref[0], (((1,), (1,)), ((), ())),
                             preferred_element_type=jnp.float32) + b2_ref[0]
        g = jnp.clip(x2, -20.0, 20.0)
        g = g / (1.0 + jnp.exp(-g))
        hid = jnp.clip(x1 * g, -10000.0, 10000.0)
        y = lax.dot_general(hid, wo_ref[0], (((1,), (1,)), ((), ())),
                            preferred_element_type=jnp.float32) + bo_ref[0]
        y_ref[...] = jnp.clip(y, -10000.0, 10000.0)


def _gmm(ntiles, te, xs, W1, W2, Wo, eng, enb, b1, b2, bo):
    def row_idx(t, nt, te):
        return (jnp.minimum(t, nt[0] - 1), 0)

    def exp_idx3(t, nt, te):
        return (te[jnp.minimum(t, nt[0] - 1)], 0, 0)

    def exp_idx2(t, nt, te):
        return (te[jnp.minimum(t, nt[0] - 1)], 0)

    grid_spec = pltpu.PrefetchScalarGridSpec(
        num_scalar_prefetch=2,
        grid=(TMAX,),
        in_specs=[
            pl.BlockSpec((BLK, D), row_idx),
            pl.BlockSpec((1, H, D), exp_idx3),
            pl.BlockSpec((1, H, D), exp_idx3),
            pl.BlockSpec((1, D, H), exp_idx3),
            pl.BlockSpec((1, 1, D), exp_idx3),
            pl.BlockSpec((1, 1, D), exp_idx3),
            pl.BlockSpec((1, 1, H), exp_idx3),
            pl.BlockSpec((1, 1, H), exp_idx3),
            pl.BlockSpec((1, 1, D), exp_idx3),
        ],
        out_specs=pl.BlockSpec((BLK, D), row_idx),
    )
    return pl.pallas_call(
        _gmm_body,
        grid_spec=grid_spec,
        out_shape=jax.ShapeDtypeStruct((P, D), jnp.float32),
        compiler_params=pltpu.CompilerParams(vmem_limit_bytes=120 * 2**20),
    )(ntiles, te, xs, W1, W2, Wo,
      eng.reshape(E, 1, D), enb.reshape(E, 1, D),
      b1.reshape(E, 1, H), b2.reshape(E, 1, H), bo.reshape(E, 1, D))


TOK_W = N // NW
CCH = 16


def _combine_body(y_hbm, dest2_hbm, z0_hbm, z1_hbm, d_m,
                  r0a, r0b, r1a, r1b, semG0, semG1, semW0, semW1):
    wid = lax.axis_index("s") * NC + lax.axis_index("c")
    nch = TOK_W // CCH
    rbase = wid * nch
    pltpu.sync_copy(dest2_hbm.at[pl.ds(rbase, nch)], d_m.at[pl.ds(0, nch)])
    pltpu.sync_copy(dest2_hbm.at[pl.ds(N // CCH + rbase, nch)],
                    d_m.at[pl.ds(nch, nch)])
    r0 = (r0a, r0b)
    r1 = (r1a, r1b)
    semG = (semG0, semG1)
    semW = (semW0, semW1)
    gt = [None] * nch
    wr = [None] * nch
    gt[0] = (pltpu.async_copy(y_hbm.at[d_m.at[0]], r0[0], semG[0]),
             pltpu.async_copy(y_hbm.at[d_m.at[nch]], r1[0], semG[0]))
    for c in range(nch):
        nb = wid * TOK_W + c * CCH
        gt[c][0].wait()
        gt[c][1].wait()
        wr[c] = (pltpu.async_copy(r0[c % 2], z0_hbm.at[pl.ds(nb, CCH)],
                                  semW[c % 2]),
                 pltpu.async_copy(r1[c % 2], z1_hbm.at[pl.ds(nb, CCH)],
                                  semW[c % 2]))
        if c + 1 < nch:
            if c >= 1:
                wr[c - 1][0].wait()
                wr[c - 1][1].wait()
            gt[c + 1] = (pltpu.async_copy(y_hbm.at[d_m.at[c + 1]],
                                          r0[(c + 1) % 2], semG[(c + 1) % 2]),
                         pltpu.async_copy(y_hbm.at[d_m.at[nch + c + 1]],
                                          r1[(c + 1) % 2], semG[(c + 1) % 2]))
    wr[nch - 2][0].wait()
    wr[nch - 2][1].wait()
    wr[nch - 1][0].wait()
    wr[nch - 1][1].wait()


def _combine_gather(y, destf):
    f = pl.kernel(
        _combine_body,
        out_type=[
            jax.ShapeDtypeStruct((N, D), jnp.float32),
            jax.ShapeDtypeStruct((N, D), jnp.float32),
        ],
        mesh=_sc_mesh(),
        scratch_types=[
            pltpu.VMEM((2 * (TOK_W // CCH), CCH), jnp.int32),
            pltpu.VMEM((CCH, D), jnp.float32),
            pltpu.VMEM((CCH, D), jnp.float32),
            pltpu.VMEM((CCH, D), jnp.float32),
            pltpu.VMEM((CCH, D), jnp.float32),
            pltpu.SemaphoreType.DMA,
            pltpu.SemaphoreType.DMA,
            pltpu.SemaphoreType.DMA,
            pltpu.SemaphoreType.DMA,
        ],
    )
    return f(y, destf.reshape(A // CCH, CCH))


def _blend_body(z0_ref, z1_ref, g0_ref, g1_ref, out_ref):
    out_ref[...] = g0_ref[...] * z0_ref[...] + g1_ref[...] * z1_ref[...]


def _blend(z0, z1, g0, g1):
    return pl.pallas_call(
        _blend_body,
        grid=(N // BN,),
        in_specs=[
            pl.BlockSpec((BN, D), lambda i: (i, 0)),
            pl.BlockSpec((BN, D), lambda i: (i, 0)),
            pl.BlockSpec((BN, 1), lambda i: (i, 0)),
            pl.BlockSpec((BN, 1), lambda i: (i, 0)),
        ],
        out_specs=pl.BlockSpec((BN, D), lambda i: (i, 0)),
        out_shape=jax.ShapeDtypeStruct((N, D), jnp.float32),
    )(z0, z1, g0, g1)


def kernel(x, ln_g, ln_b, Wr, eng, enb, W1, b1, W2, b2, Wo, bo):
    assert x.shape == (1, N, D)
    xf = x.reshape(N, D)
    ln2, idx, gate = _router(xf, ln_g.reshape(1, D), ln_b.reshape(1, D), Wr)
    idxT = idx.T.reshape(GR, AW)
    dest2, te2, nt2 = _plan(idxT)
    destf = dest2.reshape(A)
    xs = _dispatch(ln2, destf)
    y = _gmm(nt2.reshape(1), te2.reshape(TE_PAD), xs,
             W1, W2, Wo, eng, enb, b1, b2, bo)
    z0, z1 = _combine_gather(y, destf)
    out = _blend(z0, z1, gate[:, 0:1], gate[:, 1:2])
    return out.reshape(x.shape)
